# baseline (speedup 1.0000x reference)
import functools

import jax
import jax.numpy as jnp
from jax import lax
from jax.experimental import pallas as pl
from jax.experimental.pallas import tpu as pltpu
from jax.experimental.pallas import tpu_sc as plsc

N = 10000
E = 320000
D_IN = 128
D_HID = 128
D_OUT = 32

NC = 2
NS = 16
NW = NC * NS
EPW = E // NW
CHUNK = 40
NCHUNK = EPW // CHUNK
CHUNK_L = 80
NCHUNK_L = EPW // CHUNK_L
K = 5
NROUND = NCHUNK // K
NROUND_L = NCHUNK_L // K
NP = 10112
RPT = NP // NS
DEG_W = 8

_mesh = plsc.VectorSubcoreMesh(core_axis_name="c", subcore_axis_name="s")
_sc_params = pltpu.CompilerParams(use_tc_tiling_on_sc=False)


@functools.partial(
    pl.kernel,
    mesh=_mesh,
    compiler_params=_sc_params,
    out_type=jax.ShapeDtypeStruct((2 * NP, 2 * DEG_W), jnp.float32),
    scratch_types=[
        pltpu.VMEM((NCHUNK_L, CHUNK_L), jnp.int32),
        pltpu.VMEM((NCHUNK_L, CHUNK_L), jnp.int32),
        pltpu.VMEM((CHUNK_L, 2 * DEG_W), jnp.float32),
        pltpu.VMEM((CHUNK_L, 2 * DEG_W), jnp.float32),
        pltpu.VMEM_SHARED((NP, 2 * DEG_W), jnp.float32),
    ] + [pltpu.SemaphoreType.DMA] * (2 * K),
)
def _deg_kernel(src_hbm, dst_hbm, ones_src_hbm, ones_dst_hbm, zeros_hbm,
                out_hbm, srcv, dstv, ones_s, ones_d, acc_deg, *sems):
    asem = sems[:K]
    bsem = sems[K:2 * K]
    c = lax.axis_index("c")
    s = lax.axis_index("s")
    wid = s * NC + c
    r0 = s * RPT
    pltpu.sync_copy(ones_src_hbm, ones_s)
    pltpu.sync_copy(ones_dst_hbm, ones_d)
    pltpu.sync_copy(src_hbm.at[wid], srcv)
    pltpu.sync_copy(dst_hbm.at[wid], dstv)
    pltpu.sync_copy(zeros_hbm.at[pl.ds(r0, RPT)], acc_deg.at[pl.ds(r0, RPT)])
    plsc.subcore_barrier()

    for b in range(K):
        pltpu.async_copy(ones_s, acc_deg.at[srcv.at[b]], asem[b], add=True)
        pltpu.async_copy(ones_d, acc_deg.at[dstv.at[b]], bsem[b], add=True)

    def body(i, carry):
        for b in range(K):
            pltpu.make_async_copy(ones_s, acc_deg.at[srcv.at[0]],
                                  asem[b]).wait()
            pltpu.make_async_copy(ones_d, acc_deg.at[dstv.at[0]],
                                  bsem[b]).wait()

        @pl.when(i + 1 < NROUND_L)
        def _():
            for b in range(K):
                j = (i + 1) * K + b
                pltpu.async_copy(ones_s, acc_deg.at[srcv.at[j]], asem[b],
                                 add=True)
                pltpu.async_copy(ones_d, acc_deg.at[dstv.at[j]], bsem[b],
                                 add=True)

        return carry

    lax.fori_loop(0, NROUND_L, body, 0)
    plsc.subcore_barrier()
    pltpu.sync_copy(acc_deg.at[pl.ds(r0, RPT)],
                    out_hbm.at[pl.ds(c * NP + r0, RPT)])


def _make_agg(D, chunk, k, epw=EPW):
    nchunk = epw // chunk
    nround = nchunk // k

    @functools.partial(
        pl.kernel,
        mesh=_mesh,
        compiler_params=_sc_params,
        out_type=jax.ShapeDtypeStruct((2 * NP, D), jnp.float32),
        scratch_types=[
            pltpu.VMEM((nchunk, chunk), jnp.int32),
            pltpu.VMEM((nchunk, chunk), jnp.int32),
            pltpu.VMEM_SHARED((NP, D), jnp.float32),
        ] + [pltpu.VMEM((chunk, D), jnp.float32)] * k
          + [pltpu.SemaphoreType.DMA] * (2 * k),
    )
    def agg(h_hbm, src_hbm, dst_hbm, zeros_hbm, out_hbm,
            srcv, dstv, acc_sh, *rest):
        rows = rest[:k]
        gsem = rest[k:2 * k]
        ssem = rest[2 * k:3 * k]
        c = lax.axis_index("c")
        s = lax.axis_index("s")
        wid = s * NC + c
        r0 = s * RPT
        pltpu.sync_copy(src_hbm.at[wid], srcv)
        pltpu.sync_copy(dst_hbm.at[wid], dstv)
        pltpu.sync_copy(zeros_hbm.at[pl.ds(r0, RPT)], acc_sh.at[pl.ds(r0, RPT)])
        plsc.subcore_barrier()

        for b in range(k):
            pltpu.async_copy(h_hbm.at[srcv.at[b]], rows[b], gsem[b])

        ngrp = 2 if k % 2 == 0 else 1
        half = k // ngrp

        def body(i, carry):
            for g in range(ngrp):
                grp = range(g * half, (g + 1) * half)
                for b in grp:
                    pltpu.make_async_copy(h_hbm.at[srcv.at[0]], rows[b],
                                          gsem[b]).wait()
                    pltpu.async_copy(rows[b], acc_sh.at[dstv.at[i * k + b]],
                                     ssem[b], add=True)

                @pl.when(i + 1 < nround)
                def _(grp=grp):
                    for b in grp:
                        pltpu.make_async_copy(rows[b], acc_sh.at[dstv.at[0]],
                                              ssem[b]).wait()
                        pltpu.async_copy(h_hbm.at[srcv.at[(i + 1) * k + b]],
                                         rows[b], gsem[b])

            return carry

        lax.fori_loop(0, nround, body, 0)
        for b in range(k):
            pltpu.make_async_copy(rows[b], acc_sh.at[dstv.at[0]],
                                  ssem[b]).wait()
        plsc.subcore_barrier()
        pltpu.sync_copy(acc_sh.at[pl.ds(r0, RPT)],
                        out_hbm.at[pl.ds(c * NP + r0, RPT)])

    return agg


EPW_P = 10240
_agg128 = _make_agg(D_HID, CHUNK, 4, epw=EPW_P)
_agg32 = _make_agg(D_OUT, CHUNK, 2 * K)


def _dot(a, b):
    return jnp.dot(a, b, precision=lax.Precision.HIGHEST,
                   preferred_element_type=jnp.float32)


BN = 2000


def _norm_mm_body(d_ref, x_ref, w_ref, h_ref, ns_ref, nd_ref):
    dsum = d_ref[0] + d_ref[1]
    od = jnp.sum(dsum[:, :DEG_W], axis=-1) * (1.0 / DEG_W)
    idg = jnp.sum(dsum[:, DEG_W:], axis=-1) * (1.0 / DEG_W)
    ns = jnp.where(od > 0.5, lax.rsqrt(jnp.maximum(od, 1e-12)), 0.0)
    nd = jnp.where(idg > 0.5, lax.rsqrt(jnp.maximum(idg, 1e-12)), 0.0)
    h_ref[...] = _dot(x_ref[...] * ns[:, None], w_ref[...])
    ns_ref[...] = ns[:, None]
    nd_ref[...] = nd[:, None]


_norm_mm = pl.pallas_call(
    _norm_mm_body,
    grid=(N // BN,),
    in_specs=[
        pl.BlockSpec((2, BN, 2 * DEG_W), lambda i: (0, i, 0)),
        pl.BlockSpec((BN, D_IN), lambda i: (i, 0)),
        pl.BlockSpec((D_IN, D_HID), lambda i: (0, 0)),
    ],
    out_specs=[
        pl.BlockSpec((BN, D_HID), lambda i: (i, 0)),
        pl.BlockSpec((BN, 1), lambda i: (i, 0)),
        pl.BlockSpec((BN, 1), lambda i: (i, 0)),
    ],
    out_shape=[
        jax.ShapeDtypeStruct((N, D_HID), jnp.float32),
        jax.ShapeDtypeStruct((N, 1), jnp.float32),
        jax.ShapeDtypeStruct((N, 1), jnp.float32),
    ],
)


def _mid_body(p_ref, w_ref, b_ref, nd_ref, ns_ref, o_ref):
    agg = p_ref[0] + p_ref[1]
    h = jnp.maximum(agg * nd_ref[...] + b_ref[...], 0.0)
    o_ref[...] = _dot(h, w_ref[...]) * ns_ref[...]


_mid = pl.pallas_call(
    _mid_body,
    grid=(N // BN,),
    in_specs=[
        pl.BlockSpec((2, BN, D_HID), lambda i: (0, i, 0)),
        pl.BlockSpec((D_HID, D_OUT), lambda i: (0, 0)),
        pl.BlockSpec((1, D_HID), lambda i: (0, 0)),
        pl.BlockSpec((BN, 1), lambda i: (i, 0)),
        pl.BlockSpec((BN, 1), lambda i: (i, 0)),
    ],
    out_specs=pl.BlockSpec((BN, D_OUT), lambda i: (i, 0)),
    out_shape=jax.ShapeDtypeStruct((N, D_OUT), jnp.float32),
)


def _final_body(p_ref, b_ref, nd_ref, o_ref):
    agg = p_ref[0, :N] + p_ref[1, :N]
    h = jnp.maximum(agg * nd_ref[...] + b_ref[...], 0.0)
    o_ref[...] = jnp.sum(h, axis=0, keepdims=True) * (1.0 / N)


_final = pl.pallas_call(
    _final_body,
    out_shape=jax.ShapeDtypeStruct((1, D_OUT), jnp.float32),
)


def kernel(x, edge_index, W1, b1, W2, b2):
    src_flat = edge_index[0].astype(jnp.int32)
    dst_flat = edge_index[1].astype(jnp.int32)
    src = src_flat.reshape(NW, NCHUNK, CHUNK)
    dst = dst_flat.reshape(NW, NCHUNK, CHUNK)
    src_l = src_flat.reshape(NW, NCHUNK_L, CHUNK_L)
    dst_l = dst_flat.reshape(NW, NCHUNK_L, CHUNK_L)
    z128 = jnp.zeros((NP, D_HID), jnp.float32)
    z32 = jnp.zeros((NP, D_OUT), jnp.float32)
    zdeg = jnp.zeros((NP, 2 * DEG_W), jnp.float32)
    ones_src = jnp.concatenate(
        [jnp.ones((CHUNK_L, DEG_W), jnp.float32),
         jnp.zeros((CHUNK_L, DEG_W), jnp.float32)], axis=1)
    ones_dst = jnp.concatenate(
        [jnp.zeros((CHUNK_L, DEG_W), jnp.float32),
         jnp.ones((CHUNK_L, DEG_W), jnp.float32)], axis=1)

    degp = _deg_kernel(src_l, dst_l, ones_src, ones_dst, zdeg)
    h1p, ns, nd = _norm_mm(degp.reshape(2, NP, 2 * DEG_W), x, W1)

    src2d = src_flat.reshape(NW, EPW)
    dst2d = dst_flat.reshape(NW, EPW)
    pad_n = EPW_P - EPW
    srcp = jnp.concatenate(
        [src2d, jnp.full((NW, pad_n), N, jnp.int32)], axis=1
    ).reshape(NW, EPW_P // CHUNK, CHUNK)
    dstp = jnp.concatenate(
        [dst2d, jnp.zeros((NW, pad_n), jnp.int32)], axis=1
    ).reshape(NW, EPW_P // CHUNK, CHUNK)
    h1pad = jnp.concatenate(
        [h1p, jnp.zeros((NP - N, D_HID), jnp.float32)], axis=0)
    parts1 = _agg128(h1pad, srcp, dstp, z128).reshape(2, NP, D_HID)
    h2p = _mid(parts1, W2, b1.reshape(1, D_HID), nd, ns)
    parts2 = _agg32(h2p, src, dst, z32).reshape(2, NP, D_OUT)
    out = _final(parts2, b2.reshape(1, D_OUT), nd)
    return out[0]


# speedup vs baseline: 2.0337x; 2.0337x over previous
import functools

import jax
import jax.numpy as jnp
from jax import lax
from jax.experimental import pallas as pl
from jax.experimental.pallas import tpu as pltpu
from jax.experimental.pallas import tpu_sc as plsc

N = 10000
E = 320000
D_IN = 128
D_HID = 128
D_OUT = 32

NC = 2
NS = 16
NW = NC * NS
EPW = E // NW
CHUNK = 40
NCHUNK = EPW // CHUNK
CHUNK_L = 80
NCHUNK_L = EPW // CHUNK_L
K = 5
NROUND = NCHUNK // K
NROUND_L = NCHUNK_L // K
EPW_D = 10240
ND_CHUNK = EPW_D // CHUNK_L
KD = 8
ND_ROUND = ND_CHUNK // KD
NP = 10112
RPT = NP // NS
DEG_W = 8

_mesh = plsc.VectorSubcoreMesh(core_axis_name="c", subcore_axis_name="s")
_sc_params = pltpu.CompilerParams(use_tc_tiling_on_sc=False)


@functools.partial(
    pl.kernel,
    mesh=_mesh,
    compiler_params=_sc_params,
    out_type=jax.ShapeDtypeStruct((2 * NP, 2 * DEG_W), jnp.float32),
    scratch_types=[
        pltpu.VMEM((ND_CHUNK, CHUNK_L), jnp.int32),
        pltpu.VMEM((ND_CHUNK, CHUNK_L), jnp.int32),
        pltpu.VMEM((CHUNK_L, 2 * DEG_W), jnp.float32),
        pltpu.VMEM((CHUNK_L, 2 * DEG_W), jnp.float32),
        pltpu.VMEM_SHARED((NP, 2 * DEG_W), jnp.float32),
    ] + [pltpu.SemaphoreType.DMA] * (2 * KD),
)
def _deg_kernel(src_hbm, dst_hbm, ones_src_hbm, ones_dst_hbm, zeros_hbm,
                out_hbm, srcv, dstv, ones_s, ones_d, acc_deg, *sems):
    asem = sems[:KD]
    bsem = sems[KD:2 * KD]
    c = lax.axis_index("c")
    s = lax.axis_index("s")
    wid = s * NC + c
    r0 = s * RPT
    pltpu.sync_copy(ones_src_hbm, ones_s)
    pltpu.sync_copy(ones_dst_hbm, ones_d)
    pltpu.sync_copy(src_hbm.at[wid], srcv)
    pltpu.sync_copy(dst_hbm.at[wid], dstv)
    pltpu.sync_copy(zeros_hbm.at[pl.ds(r0, RPT)], acc_deg.at[pl.ds(r0, RPT)])
    plsc.subcore_barrier()

    for b in range(KD):
        pltpu.async_copy(ones_s, acc_deg.at[srcv.at[b]], asem[b], add=True)
        pltpu.async_copy(ones_d, acc_deg.at[dstv.at[b]], bsem[b], add=True)

    def body(i, carry):
        for g in range(2):
            grp = range(g * (KD // 2), (g + 1) * (KD // 2))
            for b in grp:
                pltpu.make_async_copy(ones_s, acc_deg.at[srcv.at[0]],
                                      asem[b]).wait()
                pltpu.make_async_copy(ones_d, acc_deg.at[dstv.at[0]],
                                      bsem[b]).wait()

            @pl.when(i + 1 < ND_ROUND)
            def _(grp=grp):
                for b in grp:
                    j = (i + 1) * KD + b
                    pltpu.async_copy(ones_s, acc_deg.at[srcv.at[j]], asem[b],
                                     add=True)
                    pltpu.async_copy(ones_d, acc_deg.at[dstv.at[j]], bsem[b],
                                     add=True)

        return carry

    lax.fori_loop(0, ND_ROUND, body, 0)
    plsc.subcore_barrier()
    pltpu.sync_copy(acc_deg.at[pl.ds(r0, RPT)],
                    out_hbm.at[pl.ds(c * NP + r0, RPT)])


def _make_agg(D, chunk, k):
    nchunk = EPW // chunk
    nround = nchunk // k

    @functools.partial(
        pl.kernel,
        mesh=_mesh,
        compiler_params=_sc_params,
        out_type=jax.ShapeDtypeStruct((2 * NP, D), jnp.float32),
        scratch_types=[
            pltpu.VMEM((nchunk, chunk), jnp.int32),
            pltpu.VMEM((nchunk, chunk), jnp.int32),
            pltpu.VMEM_SHARED((NP, D), jnp.float32),
        ] + [pltpu.VMEM((chunk, D), jnp.float32)] * k
          + [pltpu.SemaphoreType.DMA] * (2 * k),
    )
    def agg(h_hbm, src_hbm, dst_hbm, zeros_hbm, out_hbm,
            srcv, dstv, acc_sh, *rest):
        rows = rest[:k]
        gsem = rest[k:2 * k]
        ssem = rest[2 * k:3 * k]
        c = lax.axis_index("c")
        s = lax.axis_index("s")
        wid = s * NC + c
        r0 = s * RPT
        pltpu.sync_copy(src_hbm.at[wid], srcv)
        pltpu.sync_copy(dst_hbm.at[wid], dstv)
        pltpu.sync_copy(zeros_hbm.at[pl.ds(r0, RPT)], acc_sh.at[pl.ds(r0, RPT)])
        plsc.subcore_barrier()

        for b in range(k):
            pltpu.async_copy(h_hbm.at[srcv.at[b]], rows[b], gsem[b])

        ngrp = 2 if k % 2 == 0 else 1
        half = k // ngrp

        def body(i, carry):
            for g in range(ngrp):
                grp = range(g * half, (g + 1) * half)
                for b in grp:
                    pltpu.make_async_copy(h_hbm.at[srcv.at[0]], rows[b],
                                          gsem[b]).wait()
                    pltpu.async_copy(rows[b], acc_sh.at[dstv.at[i * k + b]],
                                     ssem[b], add=True)

                @pl.when(i + 1 < nround)
                def _(grp=grp):
                    for b in grp:
                        pltpu.make_async_copy(rows[b], acc_sh.at[dstv.at[0]],
                                              ssem[b]).wait()
                        pltpu.async_copy(h_hbm.at[srcv.at[(i + 1) * k + b]],
                                         rows[b], gsem[b])

            return carry

        lax.fori_loop(0, nround, body, 0)
        for b in range(k):
            pltpu.make_async_copy(rows[b], acc_sh.at[dstv.at[0]],
                                  ssem[b]).wait()
        plsc.subcore_barrier()
        pltpu.sync_copy(acc_sh.at[pl.ds(r0, RPT)],
                        out_hbm.at[pl.ds(c * NP + r0, RPT)])

    return agg


_agg128 = _make_agg(D_HID, CHUNK, K)
_agg32 = _make_agg(D_OUT, CHUNK, 2 * K)


def _dot(a, b):
    return jnp.dot(a, b, precision=lax.Precision.HIGHEST,
                   preferred_element_type=jnp.float32)


BN = 2000


def _norm_mm_body(d_ref, x_ref, w_ref, h_ref, ns_ref, nd_ref):
    dsum = d_ref[0] + d_ref[1]
    od = jnp.sum(dsum[:, :DEG_W], axis=-1) * (1.0 / DEG_W)
    idg = jnp.sum(dsum[:, DEG_W:], axis=-1) * (1.0 / DEG_W)
    ns = jnp.where(od > 0.5, lax.rsqrt(jnp.maximum(od, 1e-12)), 0.0)
    nd = jnp.where(idg > 0.5, lax.rsqrt(jnp.maximum(idg, 1e-12)), 0.0)
    h_ref[...] = _dot(x_ref[...] * ns[:, None], w_ref[...])
    ns_ref[...] = ns[:, None]
    nd_ref[...] = nd[:, None]


_norm_mm = pl.pallas_call(
    _norm_mm_body,
    grid=(N // BN,),
    in_specs=[
        pl.BlockSpec((2, BN, 2 * DEG_W), lambda i: (0, i, 0)),
        pl.BlockSpec((BN, D_IN), lambda i: (i, 0)),
        pl.BlockSpec((D_IN, D_HID), lambda i: (0, 0)),
    ],
    out_specs=[
        pl.BlockSpec((BN, D_HID), lambda i: (i, 0)),
        pl.BlockSpec((BN, 1), lambda i: (i, 0)),
        pl.BlockSpec((BN, 1), lambda i: (i, 0)),
    ],
    out_shape=[
        jax.ShapeDtypeStruct((N, D_HID), jnp.float32),
        jax.ShapeDtypeStruct((N, 1), jnp.float32),
        jax.ShapeDtypeStruct((N, 1), jnp.float32),
    ],
)


def _mid_body(p_ref, w_ref, b_ref, nd_ref, ns_ref, o_ref):
    agg = p_ref[0] + p_ref[1]
    h = jnp.maximum(agg * nd_ref[...] + b_ref[...], 0.0)
    o_ref[...] = _dot(h, w_ref[...]) * ns_ref[...]


_mid = pl.pallas_call(
    _mid_body,
    grid=(N // BN,),
    in_specs=[
        pl.BlockSpec((2, BN, D_HID), lambda i: (0, i, 0)),
        pl.BlockSpec((D_HID, D_OUT), lambda i: (0, 0)),
        pl.BlockSpec((1, D_HID), lambda i: (0, 0)),
        pl.BlockSpec((BN, 1), lambda i: (i, 0)),
        pl.BlockSpec((BN, 1), lambda i: (i, 0)),
    ],
    out_specs=pl.BlockSpec((BN, D_OUT), lambda i: (i, 0)),
    out_shape=jax.ShapeDtypeStruct((N, D_OUT), jnp.float32),
)


def _final_body(p_ref, b_ref, nd_ref, o_ref):
    agg = p_ref[0, :N] + p_ref[1, :N]
    h = jnp.maximum(agg * nd_ref[...] + b_ref[...], 0.0)
    o_ref[...] = jnp.sum(h, axis=0, keepdims=True) * (1.0 / N)


_final = pl.pallas_call(
    _final_body,
    out_shape=jax.ShapeDtypeStruct((1, D_OUT), jnp.float32),
)


def kernel(x, edge_index, W1, b1, W2, b2):
    src_flat = edge_index[0].astype(jnp.int32)
    dst_flat = edge_index[1].astype(jnp.int32)
    src = src_flat.reshape(NW, NCHUNK, CHUNK)
    dst = dst_flat.reshape(NW, NCHUNK, CHUNK)
    src_l = src_flat.reshape(NW, NCHUNK_L, CHUNK_L)
    dst_l = dst_flat.reshape(NW, NCHUNK_L, CHUNK_L)
    z128 = jnp.zeros((NP, D_HID), jnp.float32)
    z32 = jnp.zeros((NP, D_OUT), jnp.float32)
    zdeg = jnp.zeros((NP, 2 * DEG_W), jnp.float32)
    ones_src = jnp.concatenate(
        [jnp.ones((CHUNK_L, DEG_W), jnp.float32),
         jnp.zeros((CHUNK_L, DEG_W), jnp.float32)], axis=1)
    ones_dst = jnp.concatenate(
        [jnp.zeros((CHUNK_L, DEG_W), jnp.float32),
         jnp.ones((CHUNK_L, DEG_W), jnp.float32)], axis=1)

    pad_d = EPW_D - EPW
    src_d = jnp.concatenate(
        [src_flat.reshape(NW, EPW), jnp.full((NW, pad_d), N, jnp.int32)],
        axis=1).reshape(NW, ND_CHUNK, CHUNK_L)
    dst_d = jnp.concatenate(
        [dst_flat.reshape(NW, EPW), jnp.full((NW, pad_d), N, jnp.int32)],
        axis=1).reshape(NW, ND_CHUNK, CHUNK_L)
    degp = _deg_kernel(src_d, dst_d, ones_src, ones_dst, zdeg)
    h1p, ns, nd = _norm_mm(degp.reshape(2, NP, 2 * DEG_W), x, W1)

    parts1 = _agg128(h1p, src, dst, z128).reshape(2, NP, D_HID)
    h2p = _mid(parts1, W2, b1.reshape(1, D_HID), nd, ns)
    parts2 = _agg32(h2p, src, dst, z32).reshape(2, NP, D_OUT)
    out = _final(parts2, b2.reshape(1, D_OUT), nd)
    return out[0]


# speedup vs baseline: 2.1157x; 1.0403x over previous
import functools

import jax
import jax.numpy as jnp
from jax import lax
from jax.experimental import pallas as pl
from jax.experimental.pallas import tpu as pltpu
from jax.experimental.pallas import tpu_sc as plsc

N = 10000
E = 320000
D_IN = 128
D_HID = 128
D_OUT = 32

NC = 2
NS = 16
NW = NC * NS
EPW = E // NW
CHUNK = 40
NCHUNK = EPW // CHUNK
CHUNK_L = 80
NCHUNK_L = EPW // CHUNK_L
K = 5
NROUND = NCHUNK // K
NROUND_L = NCHUNK_L // K
NP = 10112
RPT = NP // NS
DEG_W = 8

_mesh = plsc.VectorSubcoreMesh(core_axis_name="c", subcore_axis_name="s")
_sc_params = pltpu.CompilerParams(use_tc_tiling_on_sc=False)


@functools.partial(
    pl.kernel,
    mesh=_mesh,
    compiler_params=_sc_params,
    out_type=jax.ShapeDtypeStruct((2 * NP, 2 * DEG_W), jnp.float32),
    scratch_types=[
        pltpu.VMEM((NCHUNK_L, CHUNK_L), jnp.int32),
        pltpu.VMEM((NCHUNK_L, CHUNK_L), jnp.int32),
        pltpu.VMEM((CHUNK_L, 2 * DEG_W), jnp.float32),
        pltpu.VMEM((CHUNK_L, 2 * DEG_W), jnp.float32),
        pltpu.VMEM_SHARED((NP, 2 * DEG_W), jnp.float32),
    ] + [pltpu.SemaphoreType.DMA] * (2 * K),
)
def _deg_kernel(src_hbm, dst_hbm, ones_src_hbm, ones_dst_hbm, zeros_hbm,
                out_hbm, srcv, dstv, ones_s, ones_d, acc_deg, *sems):
    asem = sems[:K]
    bsem = sems[K:2 * K]
    c = lax.axis_index("c")
    s = lax.axis_index("s")
    wid = s * NC + c
    r0 = s * RPT
    pltpu.sync_copy(ones_src_hbm, ones_s)
    pltpu.sync_copy(ones_dst_hbm, ones_d)
    pltpu.sync_copy(src_hbm.at[wid], srcv)
    pltpu.sync_copy(dst_hbm.at[wid], dstv)
    pltpu.sync_copy(zeros_hbm.at[pl.ds(r0, RPT)], acc_deg.at[pl.ds(r0, RPT)])
    plsc.subcore_barrier()

    for b in range(K):
        pltpu.async_copy(ones_s, acc_deg.at[srcv.at[b]], asem[b], add=True)
        pltpu.async_copy(ones_d, acc_deg.at[dstv.at[b]], bsem[b], add=True)

    def body(i, carry):
        for b in range(K):
            pltpu.make_async_copy(ones_s, acc_deg.at[srcv.at[0]],
                                  asem[b]).wait()
            pltpu.make_async_copy(ones_d, acc_deg.at[dstv.at[0]],
                                  bsem[b]).wait()

        @pl.when(i + 1 < NROUND_L)
        def _():
            for b in range(K):
                j = (i + 1) * K + b
                pltpu.async_copy(ones_s, acc_deg.at[srcv.at[j]], asem[b],
                                 add=True)
                pltpu.async_copy(ones_d, acc_deg.at[dstv.at[j]], bsem[b],
                                 add=True)

        return carry

    lax.fori_loop(0, NROUND_L, body, 0)
    plsc.subcore_barrier()
    pltpu.sync_copy(acc_deg.at[pl.ds(r0, RPT)],
                    out_hbm.at[pl.ds(c * NP + r0, RPT)])


def _make_agg(D, chunk, k):
    nchunk = EPW // chunk
    nround = nchunk // k

    @functools.partial(
        pl.kernel,
        mesh=_mesh,
        compiler_params=_sc_params,
        out_type=jax.ShapeDtypeStruct((2 * NP, D), jnp.float32),
        scratch_types=[
            pltpu.VMEM((nchunk, chunk), jnp.int32),
            pltpu.VMEM((nchunk, chunk), jnp.int32),
            pltpu.VMEM_SHARED((NP, D), jnp.float32),
        ] + [pltpu.VMEM((chunk, D), jnp.float32)] * k
          + [pltpu.SemaphoreType.DMA] * (2 * k),
    )
    def agg(h_hbm, src_hbm, dst_hbm, zeros_hbm, out_hbm,
            srcv, dstv, acc_sh, *rest):
        rows = rest[:k]
        gsem = rest[k:2 * k]
        ssem = rest[2 * k:3 * k]
        c = lax.axis_index("c")
        s = lax.axis_index("s")
        wid = s * NC + c
        r0 = s * RPT
        pltpu.sync_copy(src_hbm.at[wid], srcv)
        pltpu.sync_copy(dst_hbm.at[wid], dstv)
        pltpu.sync_copy(zeros_hbm.at[pl.ds(r0, RPT)], acc_sh.at[pl.ds(r0, RPT)])
        plsc.subcore_barrier()

        for b in range(k):
            pltpu.async_copy(h_hbm.at[srcv.at[b]], rows[b], gsem[b])

        ngrp = 2 if k % 2 == 0 else 1
        half = k // ngrp

        def body(i, carry):
            for g in range(ngrp):
                grp = range(g * half, (g + 1) * half)
                for b in grp:
                    pltpu.make_async_copy(h_hbm.at[srcv.at[0]], rows[b],
                                          gsem[b]).wait()
                    pltpu.async_copy(rows[b], acc_sh.at[dstv.at[i * k + b]],
                                     ssem[b], add=True)

                @pl.when(i + 1 < nround)
                def _(grp=grp):
                    for b in grp:
                        pltpu.make_async_copy(rows[b], acc_sh.at[dstv.at[0]],
                                              ssem[b]).wait()
                        pltpu.async_copy(h_hbm.at[srcv.at[(i + 1) * k + b]],
                                         rows[b], gsem[b])

            return carry

        lax.fori_loop(0, nround, body, 0)
        for b in range(k):
            pltpu.make_async_copy(rows[b], acc_sh.at[dstv.at[0]],
                                  ssem[b]).wait()
        plsc.subcore_barrier()
        pltpu.sync_copy(acc_sh.at[pl.ds(r0, RPT)],
                        out_hbm.at[pl.ds(c * NP + r0, RPT)])

    return agg


_agg128 = _make_agg(D_HID, CHUNK, K)
_agg32 = _make_agg(D_OUT, CHUNK, 2 * K)


def _dot(a, b):
    return jnp.dot(a, b, precision=lax.Precision.HIGHEST,
                   preferred_element_type=jnp.float32)


BN = 2000


def _norm_mm_body(d_ref, x_ref, w_ref, h_ref, ns_ref, nd_ref):
    dsum = d_ref[0] + d_ref[1]
    od = jnp.sum(dsum[:, :DEG_W], axis=-1) * (1.0 / DEG_W)
    idg = jnp.sum(dsum[:, DEG_W:], axis=-1) * (1.0 / DEG_W)
    ns = jnp.where(od > 0.5, lax.rsqrt(jnp.maximum(od, 1e-12)), 0.0)
    nd = jnp.where(idg > 0.5, lax.rsqrt(jnp.maximum(idg, 1e-12)), 0.0)
    h_ref[...] = _dot(x_ref[...] * ns[:, None], w_ref[...])
    ns_ref[...] = ns[:, None]
    nd_ref[...] = nd[:, None]


_norm_mm = pl.pallas_call(
    _norm_mm_body,
    grid=(N // BN,),
    in_specs=[
        pl.BlockSpec((2, BN, 2 * DEG_W), lambda i: (0, i, 0)),
        pl.BlockSpec((BN, D_IN), lambda i: (i, 0)),
        pl.BlockSpec((D_IN, D_HID), lambda i: (0, 0)),
    ],
    out_specs=[
        pl.BlockSpec((BN, D_HID), lambda i: (i, 0)),
        pl.BlockSpec((BN, 1), lambda i: (i, 0)),
        pl.BlockSpec((BN, 1), lambda i: (i, 0)),
    ],
    out_shape=[
        jax.ShapeDtypeStruct((N, D_HID), jnp.float32),
        jax.ShapeDtypeStruct((N, 1), jnp.float32),
        jax.ShapeDtypeStruct((N, 1), jnp.float32),
    ],
)


def _mid_body(p_ref, w_ref, b_ref, nd_ref, ns_ref, o_ref):
    agg = p_ref[0] + p_ref[1]
    h = jnp.maximum(agg * nd_ref[...] + b_ref[...], 0.0)
    o_ref[...] = _dot(h, w_ref[...]) * ns_ref[...]


_mid = pl.pallas_call(
    _mid_body,
    grid=(N // BN,),
    in_specs=[
        pl.BlockSpec((2, BN, D_HID), lambda i: (0, i, 0)),
        pl.BlockSpec((D_HID, D_OUT), lambda i: (0, 0)),
        pl.BlockSpec((1, D_HID), lambda i: (0, 0)),
        pl.BlockSpec((BN, 1), lambda i: (i, 0)),
        pl.BlockSpec((BN, 1), lambda i: (i, 0)),
    ],
    out_specs=pl.BlockSpec((BN, D_OUT), lambda i: (i, 0)),
    out_shape=jax.ShapeDtypeStruct((N, D_OUT), jnp.float32),
)


def _final_body(p_ref, b_ref, nd_ref, o_ref):
    agg = p_ref[0, :N] + p_ref[1, :N]
    h = jnp.maximum(agg * nd_ref[...] + b_ref[...], 0.0)
    o_ref[...] = jnp.sum(h, axis=0, keepdims=True) * (1.0 / N)


_final = pl.pallas_call(
    _final_body,
    out_shape=jax.ShapeDtypeStruct((1, D_OUT), jnp.float32),
)


def kernel(x, edge_index, W1, b1, W2, b2):
    src_flat = edge_index[0].astype(jnp.int32)
    dst_flat = edge_index[1].astype(jnp.int32)
    src = src_flat.reshape(NW, NCHUNK, CHUNK)
    dst = dst_flat.reshape(NW, NCHUNK, CHUNK)
    src_l = src_flat.reshape(NW, NCHUNK_L, CHUNK_L)
    dst_l = dst_flat.reshape(NW, NCHUNK_L, CHUNK_L)
    z128 = jnp.zeros((NP, D_HID), jnp.float32)
    z32 = jnp.zeros((NP, D_OUT), jnp.float32)
    zdeg = jnp.zeros((NP, 2 * DEG_W), jnp.float32)
    ones_src = jnp.concatenate(
        [jnp.ones((CHUNK_L, DEG_W), jnp.float32),
         jnp.zeros((CHUNK_L, DEG_W), jnp.float32)], axis=1)
    ones_dst = jnp.concatenate(
        [jnp.zeros((CHUNK_L, DEG_W), jnp.float32),
         jnp.ones((CHUNK_L, DEG_W), jnp.float32)], axis=1)

    degp = _deg_kernel(src_l, dst_l, ones_src, ones_dst, zdeg)
    h1p, ns, nd = _norm_mm(degp.reshape(2, NP, 2 * DEG_W), x, W1)

    parts1 = _agg128(h1p, src, dst, z128).reshape(2, NP, D_HID)
    h2p = _mid(parts1, W2, b1.reshape(1, D_HID), nd, ns)
    parts2 = _agg32(h2p, src, dst, z32).reshape(2, NP, D_OUT)
    out = _final(parts2, b2.reshape(1, D_OUT), nd)
    return out[0]


# speedup vs baseline: 2.1381x; 1.0106x over previous
import functools

import jax
import jax.numpy as jnp
from jax import lax
from jax.experimental import pallas as pl
from jax.experimental.pallas import tpu as pltpu
from jax.experimental.pallas import tpu_sc as plsc

N = 10000
E = 320000
D_IN = 128
D_HID = 128
D_OUT = 32

NC = 2
NS = 16
NW = NC * NS
EPW = E // NW
CHUNK = 40
NCHUNK = EPW // CHUNK
CHUNK_L = 80
NCHUNK_L = EPW // CHUNK_L
K = 5
NROUND = NCHUNK // K
NROUND_L = NCHUNK_L // K
NP = 10112
RPT = NP // NS
DEG_W = 8

_mesh = plsc.VectorSubcoreMesh(core_axis_name="c", subcore_axis_name="s")
_sc_params = pltpu.CompilerParams(use_tc_tiling_on_sc=False)


@functools.partial(
    pl.kernel,
    mesh=_mesh,
    compiler_params=_sc_params,
    out_type=jax.ShapeDtypeStruct((2 * NP, 2 * DEG_W), jnp.float32),
    scratch_types=[
        pltpu.VMEM((NCHUNK_L, CHUNK_L), jnp.int32),
        pltpu.VMEM((NCHUNK_L, CHUNK_L), jnp.int32),
        pltpu.VMEM((CHUNK_L, 2 * DEG_W), jnp.float32),
        pltpu.VMEM((CHUNK_L, 2 * DEG_W), jnp.float32),
        pltpu.VMEM_SHARED((NP, 2 * DEG_W), jnp.float32),
    ] + [pltpu.SemaphoreType.DMA] * (2 * K),
)
def _deg_kernel(src_hbm, dst_hbm, ones_src_hbm, ones_dst_hbm, zeros_hbm,
                out_hbm, srcv, dstv, ones_s, ones_d, acc_deg, *sems):
    asem = sems[:K]
    bsem = sems[K:2 * K]
    c = lax.axis_index("c")
    s = lax.axis_index("s")
    wid = s * NC + c
    r0 = s * RPT
    pltpu.sync_copy(ones_src_hbm, ones_s)
    pltpu.sync_copy(ones_dst_hbm, ones_d)
    pltpu.sync_copy(src_hbm.at[wid], srcv)
    pltpu.sync_copy(dst_hbm.at[wid], dstv)
    pltpu.sync_copy(zeros_hbm.at[pl.ds(r0, RPT)], acc_deg.at[pl.ds(r0, RPT)])
    plsc.subcore_barrier()

    for b in range(K):
        pltpu.async_copy(ones_s, acc_deg.at[srcv.at[b]], asem[b], add=True)
        pltpu.async_copy(ones_d, acc_deg.at[dstv.at[b]], bsem[b], add=True)

    def body(i, carry):
        for b in range(K):
            pltpu.make_async_copy(ones_s, acc_deg.at[srcv.at[0]],
                                  asem[b]).wait()
            pltpu.make_async_copy(ones_d, acc_deg.at[dstv.at[0]],
                                  bsem[b]).wait()

        @pl.when(i + 1 < NROUND_L)
        def _():
            for b in range(K):
                j = (i + 1) * K + b
                pltpu.async_copy(ones_s, acc_deg.at[srcv.at[j]], asem[b],
                                 add=True)
                pltpu.async_copy(ones_d, acc_deg.at[dstv.at[j]], bsem[b],
                                 add=True)

        return carry

    lax.fori_loop(0, NROUND_L, body, 0)
    plsc.subcore_barrier()
    pltpu.sync_copy(acc_deg.at[pl.ds(r0, RPT)],
                    out_hbm.at[pl.ds(c * NP + r0, RPT)])


def _make_agg(D, chunk, k):
    nchunk = EPW // chunk
    nround = nchunk // k

    @functools.partial(
        pl.kernel,
        mesh=_mesh,
        compiler_params=_sc_params,
        out_type=jax.ShapeDtypeStruct((2 * NP, D), jnp.float32),
        scratch_types=[
            pltpu.VMEM((nchunk, chunk), jnp.int32),
            pltpu.VMEM((nchunk, chunk), jnp.int32),
            pltpu.VMEM_SHARED((NP, D), jnp.float32),
        ] + [pltpu.VMEM((chunk, D), jnp.float32)] * k
          + [pltpu.SemaphoreType.DMA] * (2 * k),
    )
    def agg(h_hbm, src_hbm, dst_hbm, zeros_hbm, out_hbm,
            srcv, dstv, acc_sh, *rest):
        rows = rest[:k]
        gsem = rest[k:2 * k]
        ssem = rest[2 * k:3 * k]
        c = lax.axis_index("c")
        s = lax.axis_index("s")
        wid = s * NC + c
        r0 = s * RPT
        pltpu.sync_copy(src_hbm.at[wid], srcv)
        pltpu.sync_copy(dst_hbm.at[wid], dstv)
        for b in range(k):
            pltpu.async_copy(h_hbm.at[srcv.at[b]], rows[b], gsem[b])
        pltpu.sync_copy(zeros_hbm.at[pl.ds(r0, RPT)], acc_sh.at[pl.ds(r0, RPT)])
        plsc.subcore_barrier()

        ngrp = 2 if k % 2 == 0 else 1
        half = k // ngrp

        def body(i, carry):
            for g in range(ngrp):
                grp = range(g * half, (g + 1) * half)
                for b in grp:
                    pltpu.make_async_copy(h_hbm.at[srcv.at[0]], rows[b],
                                          gsem[b]).wait()
                    pltpu.async_copy(rows[b], acc_sh.at[dstv.at[i * k + b]],
                                     ssem[b], add=True)

                @pl.when(i + 1 < nround)
                def _(grp=grp):
                    for b in grp:
                        pltpu.make_async_copy(rows[b], acc_sh.at[dstv.at[0]],
                                              ssem[b]).wait()
                        pltpu.async_copy(h_hbm.at[srcv.at[(i + 1) * k + b]],
                                         rows[b], gsem[b])

            return carry

        lax.fori_loop(0, nround, body, 0)
        for b in range(k):
            pltpu.make_async_copy(rows[b], acc_sh.at[dstv.at[0]],
                                  ssem[b]).wait()
        plsc.subcore_barrier()
        pltpu.sync_copy(acc_sh.at[pl.ds(r0, RPT)],
                        out_hbm.at[pl.ds(c * NP + r0, RPT)])

    return agg


_agg128 = _make_agg(D_HID, CHUNK, K)
_agg32 = _make_agg(D_OUT, CHUNK, 2 * K)


def _dot(a, b):
    return jnp.dot(a, b, precision=lax.Precision.HIGHEST,
                   preferred_element_type=jnp.float32)


BN = 2000


def _norm_mm_body(d_ref, x_ref, w_ref, h_ref, ns_ref, nd_ref):
    dsum = d_ref[0] + d_ref[1]
    od = jnp.sum(dsum[:, :DEG_W], axis=-1) * (1.0 / DEG_W)
    idg = jnp.sum(dsum[:, DEG_W:], axis=-1) * (1.0 / DEG_W)
    ns = jnp.where(od > 0.5, lax.rsqrt(jnp.maximum(od, 1e-12)), 0.0)
    nd = jnp.where(idg > 0.5, lax.rsqrt(jnp.maximum(idg, 1e-12)), 0.0)
    h_ref[...] = _dot(x_ref[...] * ns[:, None], w_ref[...])
    ns_ref[...] = ns[:, None]
    nd_ref[...] = nd[:, None]


_norm_mm = pl.pallas_call(
    _norm_mm_body,
    grid=(N // BN,),
    in_specs=[
        pl.BlockSpec((2, BN, 2 * DEG_W), lambda i: (0, i, 0)),
        pl.BlockSpec((BN, D_IN), lambda i: (i, 0)),
        pl.BlockSpec((D_IN, D_HID), lambda i: (0, 0)),
    ],
    out_specs=[
        pl.BlockSpec((BN, D_HID), lambda i: (i, 0)),
        pl.BlockSpec((BN, 1), lambda i: (i, 0)),
        pl.BlockSpec((BN, 1), lambda i: (i, 0)),
    ],
    out_shape=[
        jax.ShapeDtypeStruct((N, D_HID), jnp.float32),
        jax.ShapeDtypeStruct((N, 1), jnp.float32),
        jax.ShapeDtypeStruct((N, 1), jnp.float32),
    ],
)


def _mid_body(p_ref, w_ref, b_ref, nd_ref, ns_ref, o_ref):
    agg = p_ref[0] + p_ref[1]
    h = jnp.maximum(agg * nd_ref[...] + b_ref[...], 0.0)
    o_ref[...] = _dot(h, w_ref[...]) * ns_ref[...]


_mid = pl.pallas_call(
    _mid_body,
    grid=(N // BN,),
    in_specs=[
        pl.BlockSpec((2, BN, D_HID), lambda i: (0, i, 0)),
        pl.BlockSpec((D_HID, D_OUT), lambda i: (0, 0)),
        pl.BlockSpec((1, D_HID), lambda i: (0, 0)),
        pl.BlockSpec((BN, 1), lambda i: (i, 0)),
        pl.BlockSpec((BN, 1), lambda i: (i, 0)),
    ],
    out_specs=pl.BlockSpec((BN, D_OUT), lambda i: (i, 0)),
    out_shape=jax.ShapeDtypeStruct((N, D_OUT), jnp.float32),
)


def _final_body(p_ref, b_ref, nd_ref, o_ref):
    agg = p_ref[0, :N] + p_ref[1, :N]
    h = jnp.maximum(agg * nd_ref[...] + b_ref[...], 0.0)
    o_ref[...] = jnp.sum(h, axis=0, keepdims=True) * (1.0 / N)


_final = pl.pallas_call(
    _final_body,
    out_shape=jax.ShapeDtypeStruct((1, D_OUT), jnp.float32),
)


def kernel(x, edge_index, W1, b1, W2, b2):
    src_flat = edge_index[0].astype(jnp.int32)
    dst_flat = edge_index[1].astype(jnp.int32)
    src = src_flat.reshape(NW, NCHUNK, CHUNK)
    dst = dst_flat.reshape(NW, NCHUNK, CHUNK)
    src_l = src_flat.reshape(NW, NCHUNK_L, CHUNK_L)
    dst_l = dst_flat.reshape(NW, NCHUNK_L, CHUNK_L)
    z128 = jnp.zeros((NP, D_HID), jnp.float32)
    z32 = jnp.zeros((NP, D_OUT), jnp.float32)
    zdeg = jnp.zeros((NP, 2 * DEG_W), jnp.float32)
    ones_src = jnp.concatenate(
        [jnp.ones((CHUNK_L, DEG_W), jnp.float32),
         jnp.zeros((CHUNK_L, DEG_W), jnp.float32)], axis=1)
    ones_dst = jnp.concatenate(
        [jnp.zeros((CHUNK_L, DEG_W), jnp.float32),
         jnp.ones((CHUNK_L, DEG_W), jnp.float32)], axis=1)

    degp = _deg_kernel(src_l, dst_l, ones_src, ones_dst, zdeg)
    h1p, ns, nd = _norm_mm(degp.reshape(2, NP, 2 * DEG_W), x, W1)

    parts1 = _agg128(h1p, src, dst, z128).reshape(2, NP, D_HID)
    h2p = _mid(parts1, W2, b1.reshape(1, D_HID), nd, ns)
    parts2 = _agg32(h2p, src, dst, z32).reshape(2, NP, D_OUT)
    out = _final(parts2, b2.reshape(1, D_OUT), nd)
    return out[0]
